# baseline (speedup 1.0000x reference)
import functools

import jax
import jax.numpy as jnp
import numpy as np
from jax import lax
from jax.experimental import pallas as pl
from jax.experimental.pallas import tpu as pltpu
from jax.experimental.pallas import tpu_sc as plsc

_SPECS = [(100000, i, 1) for i in range(20)] + [(100000, 20 + 10 * j, 10) for j in range(6)]
_NUM_TOKENS = 8
_CHUNK = 208
_NF = 80
_NSEG = 26
_D = 64
_B = 4096
_NW = 32
_RPW = _B // _NW
_V = 100001
_QPS = 25088
_PCH = 12544

_COL_TABLE = np.zeros((_NF,), dtype=np.int32)
for _i, (_, _off, _ln) in enumerate(_SPECS):
    _COL_TABLE[_off:_off + _ln] = _i


def _tr_body(a_ref, b_ref, c_ref, d_ref, out_ref):
    f32 = jnp.float32
    r = lax.broadcasted_iota(jnp.int32, (256, 256), 0)
    c = lax.broadcasted_iota(jnp.int32, (256, 256), 1)
    cm = c % 128
    perm = ((r // _D == cm // 32)
            & (r % _D == cm % 32 + 32 * (c // 128))).astype(f32)
    s = jnp.concatenate([a_ref[0], b_ref[0], c_ref[0], d_ref[0]], axis=0)
    s = jnp.where(jnp.isfinite(s), s, 0.0)
    y = lax.dot_general(s, perm, (((0,), (0,)), ((), ())),
                        preferred_element_type=f32)
    blo = lax.bitcast_convert_type(y[:, :128], jnp.int32)
    bhi = lax.bitcast_convert_type(y[:, 128:], jnp.int32)
    rlo = blo + 0x7FFF + ((blo >> 16) & 1)
    rhi = bhi + 0x7FFF + ((bhi >> 16) & 1)
    out_ref[0] = ((rlo >> 16) & 0xFFFF) | (rhi & jnp.int32(-65536))


def _run_transpose(t2, slab0, nslab):
    grid_p = _QPS // _PCH
    return pl.pallas_call(
        _tr_body,
        grid=(nslab, grid_p),
        in_specs=[
            pl.BlockSpec((1, _D, _PCH), lambda i, c: (i + slab0, 0, c)),
            pl.BlockSpec((1, _D, _PCH), lambda i, c: (i + slab0, 0, c + grid_p)),
            pl.BlockSpec((1, _D, _PCH), lambda i, c: (i + slab0, 0, c + 2 * grid_p)),
            pl.BlockSpec((1, _D, _PCH), lambda i, c: (i + slab0, 0, c + 3 * grid_p)),
        ],
        out_specs=pl.BlockSpec((1, _PCH, 128), lambda i, c: (i, c, 0)),
        out_shape=jax.ShapeDtypeStruct((nslab, _QPS, 128), jnp.int32),
        compiler_params=pltpu.CompilerParams(
            dimension_semantics=("arbitrary", "arbitrary"),
        ),
    )(t2, t2, t2, t2)


def _make_sc_body(variant):
    if variant == "A":
        nl, rpb = 16, 7
    else:
        nl, rpb = 80, 6
    nq = nl // 16

    def body(idx_hbm, tab_hbm, out_hbm, idx_v, pidx_v, win_v, obuf_v, sem, semo):
        wid = lax.axis_index("s") * 2 + lax.axis_index("c")
        b0 = wid * _RPW
        pltpu.sync_copy(idx_hbm.at[pl.ds(b0, _RPW)], idx_v)
        lane = lax.iota(jnp.int32, 16)

        def issue(r, buf):
            for q in range(nq):
                vec = idx_v[r, pl.ds(16 * q, 16)] >> 2
                if q == 0:
                    if variant == "A":
                        vec = jnp.where(lane < 14, vec, 0)
                    else:
                        vec = jnp.where(lane >= 14, vec, 0)
                pidx_v[buf, pl.ds(16 * q, 16)] = vec
            pltpu.async_copy(tab_hbm.at[pidx_v.at[buf]], win_v.at[buf], sem)

        def wait_buf(buf):
            pltpu.make_async_copy(tab_hbm.at[pidx_v.at[buf]], win_v.at[buf],
                                  sem).wait()

        def halves(buf, c, h):
            base = h * 32
            lo0 = win_v[buf, c, pl.ds(base, 16)]
            lo1 = win_v[buf, c, pl.ds(base + 16, 16)]
            return [
                lax.bitcast_convert_type(lo0 << 16, jnp.float32),
                lax.bitcast_convert_type(lo1 << 16, jnp.float32),
                lax.bitcast_convert_type(lo0 & -65536, jnp.float32),
                lax.bitcast_convert_type(lo1 & -65536, jnp.float32),
            ]

        def put(rr, srel, vals):
            for m in range(4):
                obuf_v[(rr * rpb) + srel // 2,
                       pl.ds((srel % 2) * _D + 16 * m, 16)] = vals[m]

        def process(r, rr, buf):
            hs = [idx_v[r, pl.ds(16 * q, 16)] & 3 for q in range(nq)]
            if variant == "A":
                for c in range(14):
                    put(rr, c, halves(buf, c, hs[0][c]))
            else:
                for c in range(14, 20):
                    put(rr, c - 14, halves(buf, c, hs[c // 16][c % 16]))
                for j in range(6):
                    accs = [None] * 4
                    for t in range(10):
                        c = 20 + 10 * j + t
                        vals = halves(buf, c, hs[c // 16][c % 16])
                        for m in range(4):
                            accs[m] = (vals[m] if accs[m] is None
                                       else accs[m] + vals[m])
                    put(rr, 6 + j, accs)

        issue(0, 0)

        def blk_step(blk, _):
            r0 = blk * 8
            for gg in range(4):
                ra = r0 + 2 * gg
                rb = ra + 1
                issue(rb, 1)
                wait_buf(0)
                process(ra, 2 * gg, 0)
                nxt = jnp.minimum(ra + 2, _RPW - 1)
                issue(nxt, 0)
                wait_buf(1)
                process(rb, 2 * gg + 1, 1)
            pltpu.sync_copy(obuf_v, out_hbm.at[pl.ds((b0 + r0) * rpb, 8 * rpb)])
            return ()

        lax.fori_loop(0, _RPW // 8, blk_step, (), unroll=False)
        wait_buf(0)

    return body, nl, rpb


def _run_sc(pk, tab, variant):
    body, nl, rpb = _make_sc_body(variant)
    mesh = plsc.VectorSubcoreMesh(core_axis_name="c", subcore_axis_name="s")
    return pl.kernel(
        body,
        mesh=mesh,
        out_type=jax.ShapeDtypeStruct((_B * rpb, 128), jnp.float32),
        scratch_types=[
            pltpu.VMEM((_RPW, 128), jnp.int32),
            pltpu.VMEM((2, nl), jnp.int32),
            pltpu.VMEM((2, nl, 128), jnp.int32),
            pltpu.VMEM((8 * rpb, 128), jnp.float32),
            pltpu.SemaphoreType.DMA,
            pltpu.SemaphoreType.DMA,
        ],
    )(pk, tab)


_BT = 256


def _tc_body(catA_ref, catB_ref, intf_ref, w1_ref, b1_ref, w2_ref, b2_ref,
             pw_ref, pb_ref, lg_ref, lb_ref, out_ref):
    f32 = jnp.float32
    cat_sum = jnp.concatenate([catA_ref[...], catB_ref[...]], axis=-1)
    xi = intf_ref[...]
    nz = (xi != 0).astype(f32)
    c_iota = lax.broadcasted_iota(jnp.int32, (_NF, _NSEG), 0)
    s_iota = lax.broadcasted_iota(jnp.int32, (_NF, _NSEG), 1)
    H = ((s_iota >= 20) & (c_iota >= 10 * s_iota - 180)
         & (c_iota < 10 * s_iota - 170)).astype(f32)
    counts = jnp.dot(nz, H, preferred_element_type=f32)
    recip = 1.0 / jnp.maximum(counts, 1.0)
    seg_of = lax.broadcasted_iota(jnp.int32, (_NSEG, 1664), 1) // _D
    E = (seg_of == lax.broadcasted_iota(jnp.int32, (_NSEG, 1664), 0)).astype(f32)
    scale = jnp.dot(recip, E, preferred_element_type=f32)
    cat = cat_sum * scale

    h = jnp.dot(cat, w1_ref[...], preferred_element_type=f32) + b1_ref[...]
    h = h * jax.nn.sigmoid(h)
    gate = jax.nn.sigmoid(jnp.dot(h, w2_ref[...], preferred_element_type=f32) + b2_ref[...])
    cat = cat * gate * 2.0

    for t in range(_NUM_TOKENS):
        xt = cat[:, _CHUNK * t:_CHUNK * (t + 1)]
        y = jnp.dot(xt, pw_ref[t], preferred_element_type=f32) + pb_ref[t]
        mu = jnp.mean(y, axis=-1, keepdims=True)
        var = jnp.mean((y - mu) ** 2, axis=-1, keepdims=True)
        out_ref[:, t, :] = (y - mu) / jnp.sqrt(var + 1e-5) * lg_ref[t] + lb_ref[t]


def _run_tc(catA, catB, int_feats, w1, b1, w2, b2, proj_w, proj_b, ln_g, ln_b):
    full = lambda shape: pl.BlockSpec(shape, lambda i: tuple(0 for _ in shape))
    return pl.pallas_call(
        _tc_body,
        grid=(_B // _BT,),
        in_specs=[
            pl.BlockSpec((_BT, 7 * 128), lambda i: (i, 0)),
            pl.BlockSpec((_BT, 6 * 128), lambda i: (i, 0)),
            pl.BlockSpec((_BT, _NF), lambda i: (i, 0)),
            full((1664, 416)),
            full((1, 416)),
            full((416, 1664)),
            full((1, 1664)),
            full((_NUM_TOKENS, _CHUNK, 256)),
            full((_NUM_TOKENS, 1, 256)),
            full((_NUM_TOKENS, 1, 256)),
            full((_NUM_TOKENS, 1, 256)),
        ],
        out_specs=pl.BlockSpec((_BT, _NUM_TOKENS, 256), lambda i: (i, 0, 0)),
        out_shape=jax.ShapeDtypeStruct((_B, _NUM_TOKENS, 256), jnp.float32),
        compiler_params=pltpu.CompilerParams(
            dimension_semantics=("arbitrary",),
        ),
    )(catA, catB, int_feats, w1, b1, w2, b2, proj_w, proj_b, ln_g, ln_b)


def _packed_indices(int_feats):
    offsets = jnp.asarray(_COL_TABLE.astype(np.int64) * (4 * _QPS), dtype=jnp.int32)
    h = int_feats // _QPS
    p = int_feats - h * _QPS
    pk = offsets[None, :] + 4 * p + h
    return jnp.pad(pk, ((0, 0), (0, 128 - _NF)))


def kernel(int_feats, tables, w1, b1, w2, b2, proj_w, proj_b, ln_g, ln_b):
    t2 = tables.transpose(0, 2, 1)
    pk = _packed_indices(int_feats)
    pkB = pk - 14 * 4 * _QPS
    tabA = _run_transpose(t2, 0, 14).reshape(14 * _QPS, 128)
    catA = _run_sc(pk, tabA, "A")
    tabB = _run_transpose(t2, 14, 12).reshape(12 * _QPS, 128)
    catB = _run_sc(pkB, tabB, "B")
    return _run_tc(
        catA.reshape(_B, 7 * 128), catB.reshape(_B, 6 * 128), int_feats, w1,
        b1.reshape(1, 416), w2, b2.reshape(1, 1664),
        proj_w, proj_b.reshape(_NUM_TOKENS, 1, 256),
        ln_g.reshape(_NUM_TOKENS, 1, 256), ln_b.reshape(_NUM_TOKENS, 1, 256),
    )


# speedup vs baseline: 5.3032x; 5.3032x over previous
import functools

import jax
import jax.numpy as jnp
import numpy as np
from jax import lax
from jax.experimental import pallas as pl
from jax.experimental.pallas import tpu as pltpu
from jax.experimental.pallas import tpu_sc as plsc

_SPECS = [(100000, i, 1) for i in range(20)] + [(100000, 20 + 10 * j, 10) for j in range(6)]
_NUM_TOKENS = 8
_CHUNK = 208
_NF = 80
_NSEG = 26
_D = 64
_B = 4096
_NW = 32
_RPW = _B // _NW
_V = 100001
_QPS = 25088
_PCH = 12544

_COL_TABLE = np.zeros((_NF,), dtype=np.int32)
for _i, (_, _off, _ln) in enumerate(_SPECS):
    _COL_TABLE[_off:_off + _ln] = _i


def _tr_body(a_ref, b_ref, c_ref, d_ref, out_ref):
    f32 = jnp.float32
    r = lax.broadcasted_iota(jnp.int32, (256, 256), 0)
    c = lax.broadcasted_iota(jnp.int32, (256, 256), 1)
    cm = c % 128
    perm = ((r // _D == cm // 32)
            & (r % _D == cm % 32 + 32 * (c // 128))).astype(f32)
    s = jnp.concatenate([a_ref[0], b_ref[0], c_ref[0], d_ref[0]], axis=0)
    s = jnp.where(jnp.isfinite(s), s, 0.0)
    y = lax.dot_general(s, perm, (((0,), (0,)), ((), ())),
                        preferred_element_type=f32)
    blo = lax.bitcast_convert_type(y[:, :128], jnp.int32)
    bhi = lax.bitcast_convert_type(y[:, 128:], jnp.int32)
    rlo = blo + 0x7FFF + ((blo >> 16) & 1)
    rhi = bhi + 0x7FFF + ((bhi >> 16) & 1)
    out_ref[0] = ((rlo >> 16) & 0xFFFF) | (rhi & jnp.int32(-65536))


def _run_transpose(t2):
    grid_p = _QPS // _PCH
    return pl.pallas_call(
        _tr_body,
        grid=(26, grid_p),
        in_specs=[
            pl.BlockSpec((1, _D, _PCH), lambda i, c: (i, 0, c)),
            pl.BlockSpec((1, _D, _PCH), lambda i, c: (i, 0, c + grid_p)),
            pl.BlockSpec((1, _D, _PCH), lambda i, c: (i, 0, c + 2 * grid_p)),
            pl.BlockSpec((1, _D, _PCH), lambda i, c: (i, 0, c + 3 * grid_p)),
        ],
        out_specs=pl.BlockSpec((1, _PCH, 128), lambda i, c: (i, c, 0)),
        out_shape=jax.ShapeDtypeStruct((26, _QPS, 128), jnp.int32),
        compiler_params=pltpu.CompilerParams(
            dimension_semantics=("arbitrary", "arbitrary"),
        ),
    )(t2, t2, t2, t2)


def _sc_body(idx_hbm, tab_hbm, out_hbm, idx_v, pidx_v, win_v, obuf_v, sem, semo):
    wid = lax.axis_index("s") * 2 + lax.axis_index("c")
    b0 = wid * _RPW
    pltpu.sync_copy(idx_hbm.at[pl.ds(b0, _RPW)], idx_v)

    def issue(r, buf):
        for q in range(5):
            pidx_v[buf, pl.ds(16 * q, 16)] = idx_v[r, pl.ds(16 * q, 16)] >> 2
        pltpu.async_copy(tab_hbm.at[pidx_v.at[buf]], win_v.at[buf], sem)

    def wait_buf(buf):
        pltpu.make_async_copy(tab_hbm.at[pidx_v.at[buf]], win_v.at[buf], sem).wait()

    def halves(buf, c, h):
        base = h * 32
        lo0 = win_v[buf, c, pl.ds(base, 16)]
        lo1 = win_v[buf, c, pl.ds(base + 16, 16)]
        return [
            lax.bitcast_convert_type(lo0 << 16, jnp.float32),
            lax.bitcast_convert_type(lo1 << 16, jnp.float32),
            lax.bitcast_convert_type(lo0 & -65536, jnp.float32),
            lax.bitcast_convert_type(lo1 & -65536, jnp.float32),
        ]

    def process(r, rr, buf):
        hs = [idx_v[r, pl.ds(16 * q, 16)] & 3 for q in range(5)]
        for c in range(20):
            h = hs[c // 16][c % 16]
            vals = halves(buf, c, h)
            for m in range(4):
                obuf_v[(rr * 13) + c // 2, pl.ds((c % 2) * _D + 16 * m, 16)] = vals[m]
        for j in range(6):
            accs = [None] * 4
            for t in range(10):
                c = 20 + 10 * j + t
                h = hs[c // 16][c % 16]
                vals = halves(buf, c, h)
                for m in range(4):
                    accs[m] = vals[m] if accs[m] is None else accs[m] + vals[m]
            s = 20 + j
            for m in range(4):
                obuf_v[(rr * 13) + s // 2, pl.ds((s % 2) * _D + 16 * m, 16)] = accs[m]

    issue(0, 0)

    def blk_step(blk, _):
        r0 = blk * 8
        for gg in range(4):
            ra = r0 + 2 * gg
            rb = ra + 1
            issue(rb, 1)
            wait_buf(0)
            process(ra, 2 * gg, 0)
            nxt = jnp.minimum(ra + 2, _RPW - 1)
            issue(nxt, 0)
            wait_buf(1)
            process(rb, 2 * gg + 1, 1)
        pltpu.sync_copy(obuf_v, out_hbm.at[pl.ds((b0 + r0) * 13, 104)])
        return ()

    lax.fori_loop(0, _RPW // 8, blk_step, (), unroll=False)
    wait_buf(0)


def _run_sc(pk, tab):
    mesh = plsc.VectorSubcoreMesh(core_axis_name="c", subcore_axis_name="s")
    return pl.kernel(
        _sc_body,
        mesh=mesh,
        out_type=jax.ShapeDtypeStruct((_B * 13, 128), jnp.float32),
        scratch_types=[
            pltpu.VMEM((_RPW, 128), jnp.int32),
            pltpu.VMEM((2, _NF), jnp.int32),
            pltpu.VMEM((2, _NF, 128), jnp.int32),
            pltpu.VMEM((104, 128), jnp.float32),
            pltpu.SemaphoreType.DMA,
            pltpu.SemaphoreType.DMA,
        ],
    )(pk, tab)


_BT = 256


def _tc_body(cat_ref, intf_ref, w1_ref, b1_ref, w2_ref, b2_ref,
             pw_ref, pb_ref, lg_ref, lb_ref, out_ref):
    f32 = jnp.float32
    cat_sum = cat_ref[...]
    xi = intf_ref[...]
    nz = (xi != 0).astype(f32)
    c_iota = lax.broadcasted_iota(jnp.int32, (_NF, _NSEG), 0)
    s_iota = lax.broadcasted_iota(jnp.int32, (_NF, _NSEG), 1)
    H = ((s_iota >= 20) & (c_iota >= 10 * s_iota - 180)
         & (c_iota < 10 * s_iota - 170)).astype(f32)
    counts = jnp.dot(nz, H, preferred_element_type=f32)
    recip = 1.0 / jnp.maximum(counts, 1.0)
    seg_of = lax.broadcasted_iota(jnp.int32, (_NSEG, 1664), 1) // _D
    E = (seg_of == lax.broadcasted_iota(jnp.int32, (_NSEG, 1664), 0)).astype(f32)
    scale = jnp.dot(recip, E, preferred_element_type=f32)
    cat = cat_sum * scale

    h = jnp.dot(cat, w1_ref[...], preferred_element_type=f32) + b1_ref[...]
    h = h * jax.nn.sigmoid(h)
    gate = jax.nn.sigmoid(jnp.dot(h, w2_ref[...], preferred_element_type=f32) + b2_ref[...])
    cat = cat * gate * 2.0

    for t in range(_NUM_TOKENS):
        xt = cat[:, _CHUNK * t:_CHUNK * (t + 1)]
        y = jnp.dot(xt, pw_ref[t], preferred_element_type=f32) + pb_ref[t]
        mu = jnp.mean(y, axis=-1, keepdims=True)
        var = jnp.mean((y - mu) ** 2, axis=-1, keepdims=True)
        out_ref[:, t, :] = (y - mu) / jnp.sqrt(var + 1e-5) * lg_ref[t] + lb_ref[t]


def _run_tc(cat2d, int_feats, w1, b1, w2, b2, proj_w, proj_b, ln_g, ln_b):
    full = lambda shape: pl.BlockSpec(shape, lambda i: tuple(0 for _ in shape))
    return pl.pallas_call(
        _tc_body,
        grid=(_B // _BT,),
        in_specs=[
            pl.BlockSpec((_BT, _NSEG * _D), lambda i: (i, 0)),
            pl.BlockSpec((_BT, _NF), lambda i: (i, 0)),
            full((1664, 416)),
            full((1, 416)),
            full((416, 1664)),
            full((1, 1664)),
            full((_NUM_TOKENS, _CHUNK, 256)),
            full((_NUM_TOKENS, 1, 256)),
            full((_NUM_TOKENS, 1, 256)),
            full((_NUM_TOKENS, 1, 256)),
        ],
        out_specs=pl.BlockSpec((_BT, _NUM_TOKENS, 256), lambda i: (i, 0, 0)),
        out_shape=jax.ShapeDtypeStruct((_B, _NUM_TOKENS, 256), jnp.float32),
        compiler_params=pltpu.CompilerParams(
            dimension_semantics=("arbitrary",),
        ),
    )(cat2d, int_feats, w1, b1, w2, b2, proj_w, proj_b, ln_g, ln_b)


def _packed_indices(int_feats):
    offsets = jnp.asarray(_COL_TABLE.astype(np.int64) * (4 * _QPS), dtype=jnp.int32)
    h = int_feats // _QPS
    p = int_feats - h * _QPS
    pk = offsets[None, :] + 4 * p + h
    return jnp.pad(pk, ((0, 0), (0, 128 - _NF)))


def kernel(int_feats, tables, w1, b1, w2, b2, proj_w, proj_b, ln_g, ln_b):
    t2 = tables.transpose(0, 2, 1)
    tab = _run_transpose(t2).reshape(26 * _QPS, 128)
    pk = _packed_indices(int_feats)
    cat_pk = _run_sc(pk, tab)
    return _run_tc(
        cat_pk.reshape(_B, _NSEG * _D), int_feats, w1,
        b1.reshape(1, 416), w2, b2.reshape(1, 1664),
        proj_w, proj_b.reshape(_NUM_TOKENS, 1, 256),
        ln_g.reshape(_NUM_TOKENS, 1, 256), ln_b.reshape(_NUM_TOKENS, 1, 256),
    )
